# Initial kernel scaffold; baseline (speedup 1.0000x reference)
import jax
import jax.numpy as jnp
from jax.experimental import pallas as pl


def kernel(edge_index, edge_type, node_emb, w1, root1, b1, w2, root2, b2):
    raise NotImplementedError("write your pallas kernel here")


# speedup vs baseline: 17.1909x; 17.1909x over previous
import functools

import jax
import jax.numpy as jnp
from jax import lax
from jax.experimental import pallas as pl
from jax.experimental.pallas import tpu as pltpu
from jax.experimental.pallas import tpu_sc as plsc

N = 10000
H = 200
R = 16
NB = 5
HB = H // NB
E = 320000
L = 16
HP = 224
HH = 112
NVH = HH // L
NC = 2
NS = 16
EC = E // NC
ET = EC // NS
ETA = E // NS
K = 80
NCHUNK = ET // K
NCHUNKA = ETA // K
CHK = 200
NCH = N // CHK

_f32 = jnp.float32
_i32 = jnp.int32

@functools.cache
def _sc_mesh():
    return plsc.VectorSubcoreMesh(
        core_axis_name="c", subcore_axis_name="s", num_cores=NC, num_subcores=NS
    )


def _counts_body(dst_hbm, typ_hbm, eye_hbm, cnt_hbm, dstv, typv, oh, obuf,
                 sem, cnt_sh):
    c = lax.axis_index("c")
    s = lax.axis_index("s")

    def zrow(i, _):
        obuf[i, :] = jnp.zeros((L,), _f32)
        return 0

    lax.fori_loop(0, CHK, zrow, 0)
    for mi in range((NCH + NS - 1) // NS):
        m = s + NS * mi

        @pl.when(m < NCH)
        def _():
            pltpu.sync_copy(obuf, cnt_sh.at[pl.ds(m * CHK, CHK)])

    plsc.subcore_barrier()

    base0 = c * EC + s * ET

    def chunk(q, _):
        base = base0 + q * K
        pltpu.sync_copy(dst_hbm.at[pl.ds(base, K)], dstv)
        pltpu.sync_copy(typ_hbm.at[pl.ds(base, K)], typv)
        pltpu.async_copy(eye_hbm.at[typv], oh, sem).wait()
        pltpu.sync_copy(oh, cnt_sh.at[dstv], add=True)
        return 0

    lax.fori_loop(0, NCHUNK, chunk, 0)
    plsc.subcore_barrier()
    for mi in range((NCH + NS - 1) // NS):
        m = s + NS * mi

        @pl.when(m < NCH)
        def _():
            pltpu.sync_copy(cnt_sh.at[pl.ds(m * CHK, CHK)], obuf)
            pltpu.sync_copy(obuf, cnt_hbm.at[c, m])


@functools.cache
def _counts_call():
    return pl.kernel(
        _counts_body,
        out_type=jax.ShapeDtypeStruct((NC, NCH, CHK, R), _f32),
        mesh=_sc_mesh(),
        scratch_types=[
            pltpu.VMEM((K,), _i32),
            pltpu.VMEM((K,), _i32),
            pltpu.VMEM((K, R), _f32),
            pltpu.VMEM((CHK, R), _f32),
            pltpu.SemaphoreType.DMA,
            pltpu.VMEM_SHARED((N, R), _f32),
        ],
        compiler_params=pltpu.CompilerParams(use_tc_tiling_on_sc=False),
    )


def _agg_body(src_hbm, typ_hbm, dst_hbm, z_hbm, nrm_hbm, agg_hbm,
              srcv, typv, dstv, zidx, segv, rows, nrmv, zbuf, sem1, sem2,
              agg_sh):
    c = lax.axis_index("c")
    s = lax.axis_index("s")

    def zrow(i, _):
        for j in range(NVH):
            zbuf[i, pl.ds(L * j, L)] = jnp.zeros((L,), _f32)
        return 0

    lax.fori_loop(0, CHK, zrow, 0)
    for mi in range((NCH + NS - 1) // NS):
        m = s + NS * mi

        @pl.when(m < NCH)
        def _():
            pltpu.sync_copy(zbuf, agg_sh.at[pl.ds(m * CHK, CHK)])

    plsc.subcore_barrier()

    base0 = s * ETA
    zoff = c * (N * R)

    def chunk(q, _):
        base = base0 + q * K
        pltpu.sync_copy(src_hbm.at[pl.ds(base, K)], srcv)
        pltpu.sync_copy(typ_hbm.at[pl.ds(base, K)], typv)
        pltpu.sync_copy(dst_hbm.at[pl.ds(base, K)], dstv)

        def mkidx(i, _):
            sv = srcv[pl.ds(L * i, L)]
            tv = typv[pl.ds(L * i, L)]
            dv = dstv[pl.ds(L * i, L)]
            zidx[pl.ds(L * i, L)] = zoff + sv * R + tv
            segv[pl.ds(L * i, L)] = dv * R + tv
            return 0

        lax.fori_loop(0, K // L, mkidx, 0)
        cp1 = pltpu.async_copy(z_hbm.at[zidx], rows, sem1)
        cp2 = pltpu.async_copy(nrm_hbm.at[segv], nrmv, sem2)
        cp2.wait()
        cp1.wait()

        def scale(k, _):
            b = nrmv[k, :]
            for j in range(NVH):
                rows[k, pl.ds(L * j, L)] = rows[k, pl.ds(L * j, L)] * b
            return 0

        lax.fori_loop(0, K, scale, 0)
        pltpu.sync_copy(rows, agg_sh.at[dstv], add=True)
        return 0

    lax.fori_loop(0, NCHUNKA, chunk, 0)
    plsc.subcore_barrier()
    for mi in range((NCH + NS - 1) // NS):
        m = s + NS * mi

        @pl.when(m < NCH)
        def _():
            pltpu.sync_copy(agg_sh.at[pl.ds(m * CHK, CHK)], zbuf)
            pltpu.sync_copy(zbuf, agg_hbm.at[c, m])


@functools.cache
def _agg_call():
    return pl.kernel(
        _agg_body,
        out_type=jax.ShapeDtypeStruct((NC, NCH, CHK, HH), _f32),
        mesh=_sc_mesh(),
        scratch_types=[
            pltpu.VMEM((K,), _i32),
            pltpu.VMEM((K,), _i32),
            pltpu.VMEM((K,), _i32),
            pltpu.VMEM((K,), _i32),
            pltpu.VMEM((K,), _i32),
            pltpu.VMEM((K, HH), _f32),
            pltpu.VMEM((K, L), _f32),
            pltpu.VMEM((CHK, HH), _f32),
            pltpu.SemaphoreType.DMA,
            pltpu.SemaphoreType.DMA,
            pltpu.VMEM_SHARED((N, HH), _f32),
        ],
        compiler_params=pltpu.CompilerParams(use_tc_tiling_on_sc=False),
    )


def _prep_body(cnt_ref, nrm_ref):
    total = cnt_ref[0] + cnt_ref[1]
    nr = 1.0 / jnp.maximum(total, 1.0)
    nrm_ref[...] = jnp.broadcast_to(nr[:, :, None], nr.shape + (L,))


def _prep_call(cnt):
    tn = 1000
    return pl.pallas_call(
        _prep_body,
        grid=(N // tn,),
        in_specs=[pl.BlockSpec((NC, tn, R), lambda n: (0, n, 0))],
        out_specs=pl.BlockSpec((tn, R, L), lambda n: (n, 0, 0)),
        out_shape=jax.ShapeDtypeStruct((N, R, L), _f32),
    )(cnt)


def _z_body(x_ref, wl_ref, wr_ref, z_ref):
    x = x_ref[...]
    z_ref[0] = jnp.dot(x, wl_ref[...], preferred_element_type=_f32)
    z_ref[1] = jnp.dot(x, wr_ref[...], preferred_element_type=_f32)


def _z_call(x, wl, wr):
    tn = 400
    return pl.pallas_call(
        _z_body,
        grid=(N // tn,),
        in_specs=[
            pl.BlockSpec((tn, H), lambda n: (n, 0)),
            pl.BlockSpec((H, R * HH), lambda n: (0, 0)),
            pl.BlockSpec((H, R * HH), lambda n: (0, 0)),
        ],
        out_specs=pl.BlockSpec((NC, tn, R * HH), lambda n: (0, n, 0)),
        out_shape=jax.ShapeDtypeStruct((NC, N, R * HH), _f32),
    )(x, wl, wr)


def _comb_body(agg_ref, x_ref, root_ref, b_ref, out_ref, *, relu):
    a = jnp.concatenate([agg_ref[0], agg_ref[1, :, : H - HH]], axis=1)
    y = a + jnp.dot(x_ref[...], root_ref[...], preferred_element_type=_f32)
    y = y + b_ref[...]
    if relu:
        y = jnp.maximum(y, 0.0)
    out_ref[...] = y


def _comb_call(agg, x, root, b, relu):
    tn = 1000
    return pl.pallas_call(
        functools.partial(_comb_body, relu=relu),
        grid=(N // tn,),
        in_specs=[
            pl.BlockSpec((NC, tn, HH), lambda n: (0, n, 0)),
            pl.BlockSpec((tn, H), lambda n: (n, 0)),
            pl.BlockSpec((H, H), lambda n: (0, 0)),
            pl.BlockSpec((1, H), lambda n: (0, 0)),
        ],
        out_specs=pl.BlockSpec((tn, H), lambda n: (n, 0)),
        out_shape=jax.ShapeDtypeStruct((N, H), _f32),
    )(agg, x, root, b)


def _expand_weights(w):
    wfull = jnp.zeros((R, H, HP), _f32)
    for bidx in range(NB):
        lo = bidx * HB
        wfull = wfull.at[:, lo:lo + HB, lo:lo + HB].set(w[:, bidx])
    wl = wfull[:, :, :HH].transpose(1, 0, 2).reshape(H, R * HH)
    wr = wfull[:, :, HH:].transpose(1, 0, 2).reshape(H, R * HH)
    return wl, wr


def _layer(x, src, typ, dst, wl, wr, root, b, normrep, relu):
    z = _z_call(x, wl, wr).reshape(NC * N * R, HH)
    agg = _agg_call()(src, typ, dst, z, normrep).reshape(NC, N, HH)
    return _comb_call(agg, x, root, b.reshape(1, H), relu)


def kernel(edge_index, edge_type, node_emb, w1, root1, b1, w2, root2, b2):
    src = edge_index[0]
    dst = edge_index[1]
    typ = edge_type
    eye = jnp.eye(R, L, dtype=_f32)
    cnt = _counts_call()(dst, typ, eye).reshape(NC, N, R)
    normrep = _prep_call(cnt).reshape(N * R, L)
    wl1, wr1 = _expand_weights(w1)
    wl2, wr2 = _expand_weights(w2)
    x1 = _layer(node_emb, src, typ, dst, wl1, wr1, root1, b1, normrep, True)
    out = _layer(x1, src, typ, dst, wl2, wr2, root2, b2, normrep, False)
    return out


# speedup vs baseline: 27.5447x; 1.6023x over previous
import functools

import jax
import jax.numpy as jnp
from jax import lax
from jax.experimental import pallas as pl
from jax.experimental.pallas import tpu as pltpu
from jax.experimental.pallas import tpu_sc as plsc

N = 10000
H = 200
R = 16
NB = 5
HB = H // NB
E = 320000
L = 16
HP = 224
HH = 112
NVH = HH // L
NC = 2
NS = 16
EC = E // NC
ET = EC // NS
ETA = E // NS
K = 80
NCHUNK = ET // K
NCHUNKA = ETA // K
CHK = 200
NCH = N // CHK

_f32 = jnp.float32
_i32 = jnp.int32

@functools.cache
def _sc_mesh():
    return plsc.VectorSubcoreMesh(
        core_axis_name="c", subcore_axis_name="s", num_cores=NC, num_subcores=NS
    )


def _counts_body(ei_hbm, eye_hbm, cnt_hbm, eb0, eb1, tp0, tp1, dt0, dt1,
                 oh0, oh1, obuf, semE0, semE1, semG0, semG1, semS0, semS1,
                 cnt_sh):
    c = lax.axis_index("c")
    s = lax.axis_index("s")
    ebufs = (eb0, eb1)
    tps = (tp0, tp1)
    dts = (dt0, dt1)
    ohs = (oh0, oh1)
    semE = (semE0, semE1)
    semG = (semG0, semG1)
    semS = (semS0, semS1)

    def zrow(i, _):
        obuf[i, :] = jnp.zeros((L,), _f32)
        return 0

    lax.fori_loop(0, CHK, zrow, 0)
    for mi in range((NCH + NS - 1) // NS):
        m = s + NS * mi

        @pl.when(m < NCH)
        def _():
            pltpu.sync_copy(obuf, cnt_sh.at[pl.ds(m * CHK, CHK)])

    plsc.subcore_barrier()

    base0 = c * EC + s * ET

    def esrc(q):
        return ei_hbm.at[:, pl.ds(base0 + q * K, K)]

    def fire_edges(q, p):
        pltpu.async_copy(esrc(q), ebufs[p], semE[p])

    def wait_edges(q, p):
        pltpu.make_async_copy(esrc(q), ebufs[p], semE[p]).wait()

    def copy_idx(p):
        eb, tp, dt = ebufs[p], tps[p], dts[p]

        def one(i, _):
            dt[pl.ds(L * i, L)] = eb[1, pl.ds(L * i, L)]
            tp[pl.ds(L * i, L)] = eb[2, pl.ds(L * i, L)]
            return 0

        lax.fori_loop(0, K // L, one, 0)

    def fire_oh(q, p):
        pltpu.async_copy(eye_hbm.at[tps[p]], ohs[p], semG[p])

    def wait_oh(q, p):
        pltpu.make_async_copy(eye_hbm.at[tps[p]], ohs[p], semG[p]).wait()

    def fire_scatter(q, p):
        pltpu.async_copy(ohs[p], cnt_sh.at[dts[p]], semS[p], add=True)

    def wait_scatter(q, p):
        pltpu.make_async_copy(ohs[p], cnt_sh.at[dts[p]], semS[p]).wait()

    fire_edges(0, 0)

    def sub(q, p):
        pn = 1 - p

        @pl.when(q < NCHUNK)
        def _():
            wait_edges(q, p)

        @pl.when(jnp.logical_and(q >= 2, q < NCHUNK))
        def _():
            wait_scatter(q - 2, p)

        @pl.when(q < NCHUNK)
        def _():
            copy_idx(p)
            fire_oh(q, p)

        @pl.when(q + 1 < NCHUNK)
        def _():
            fire_edges(q + 1, pn)

        @pl.when(jnp.logical_and(q >= 1, q - 1 < NCHUNK))
        def _():
            wait_oh(q - 1, pn)
            fire_scatter(q - 1, pn)

    def pair(i, _):
        sub(2 * i, 0)
        sub(2 * i + 1, 1)
        return 0

    lax.fori_loop(0, (NCHUNK + 2) // 2, pair, 0)
    wait_scatter(NCHUNK - 2, (NCHUNK - 2) % 2)
    wait_scatter(NCHUNK - 1, (NCHUNK - 1) % 2)
    plsc.subcore_barrier()
    for mi in range((NCH + NS - 1) // NS):
        m = s + NS * mi

        @pl.when(m < NCH)
        def _():
            pltpu.sync_copy(cnt_sh.at[pl.ds(m * CHK, CHK)], obuf)
            pltpu.sync_copy(obuf, cnt_hbm.at[c, m])


@functools.cache
def _counts_call():
    return pl.kernel(
        _counts_body,
        out_type=jax.ShapeDtypeStruct((NC, NCH, CHK, R), _f32),
        mesh=_sc_mesh(),
        scratch_types=[
            pltpu.VMEM((3, K), _i32),
            pltpu.VMEM((3, K), _i32),
            pltpu.VMEM((K,), _i32),
            pltpu.VMEM((K,), _i32),
            pltpu.VMEM((K,), _i32),
            pltpu.VMEM((K,), _i32),
            pltpu.VMEM((K, R), _f32),
            pltpu.VMEM((K, R), _f32),
            pltpu.VMEM((CHK, R), _f32),
            pltpu.SemaphoreType.DMA,
            pltpu.SemaphoreType.DMA,
            pltpu.SemaphoreType.DMA,
            pltpu.SemaphoreType.DMA,
            pltpu.SemaphoreType.DMA,
            pltpu.SemaphoreType.DMA,
            pltpu.VMEM_SHARED((N, R), _f32),
        ],
        compiler_params=pltpu.CompilerParams(use_tc_tiling_on_sc=False),
    )


def _agg_body(ei_hbm, z_hbm, nrm_hbm, agg_hbm,
              eb0, eb1, zx0, zx1, sg0, sg1, dv0, dv1, rw0, rw1, nm0, nm1,
              zbuf, semE0, semE1, semG0, semG1, semS0, semS1, agg_sh):
    c = lax.axis_index("c")
    s = lax.axis_index("s")
    ebufs = (eb0, eb1)
    zidxs = (zx0, zx1)
    segvs = (sg0, sg1)
    dsts = (dv0, dv1)
    rws = (rw0, rw1)
    nms = (nm0, nm1)
    semE = (semE0, semE1)
    semG = (semG0, semG1)
    semS = (semS0, semS1)

    def zrow(i, _):
        for j in range(NVH):
            zbuf[i, pl.ds(L * j, L)] = jnp.zeros((L,), _f32)
        return 0

    lax.fori_loop(0, CHK, zrow, 0)
    for mi in range((NCH + NS - 1) // NS):
        m = s + NS * mi

        @pl.when(m < NCH)
        def _():
            pltpu.sync_copy(zbuf, agg_sh.at[pl.ds(m * CHK, CHK)])

    plsc.subcore_barrier()

    base0 = s * ETA
    zoff = c * (N * R)

    def esrc(q):
        return ei_hbm.at[:, pl.ds(base0 + q * K, K)]

    def fire_edges(q, p):
        pltpu.async_copy(esrc(q), ebufs[p], semE[p])

    def wait_edges(q, p):
        pltpu.make_async_copy(esrc(q), ebufs[p], semE[p]).wait()

    def mkidx(p):
        eb, zidx, segv, dstv = ebufs[p], zidxs[p], segvs[p], dsts[p]

        def one(i, _):
            sv = eb[0, pl.ds(L * i, L)]
            dv = eb[1, pl.ds(L * i, L)]
            tv = eb[2, pl.ds(L * i, L)]
            zidx[pl.ds(L * i, L)] = zoff + sv * R + tv
            segv[pl.ds(L * i, L)] = dv * R + tv
            dstv[pl.ds(L * i, L)] = dv
            return 0

        lax.fori_loop(0, K // L, one, 0)

    def fire_gathers(q, p):
        pltpu.async_copy(z_hbm.at[zidxs[p]], rws[p], semG[p])
        pltpu.async_copy(nrm_hbm.at[segvs[p]], nms[p], semG[p])

    def wait_gathers(q, p):
        pltpu.make_async_copy(z_hbm.at[zidxs[p]], rws[p], semG[p]).wait()
        pltpu.make_async_copy(nrm_hbm.at[segvs[p]], nms[p], semG[p]).wait()

    def scale(p):
        rows, nrmv = rws[p], nms[p]

        def one(k, _):
            b = nrmv[k, :]
            for j in range(NVH):
                rows[k, pl.ds(L * j, L)] = rows[k, pl.ds(L * j, L)] * b
            return 0

        lax.fori_loop(0, K, one, 0)

    def fire_scatter(q, p):
        pltpu.async_copy(rws[p], agg_sh.at[dsts[p]], semS[p], add=True)

    def wait_scatter(q, p):
        pltpu.make_async_copy(rws[p], agg_sh.at[dsts[p]], semS[p]).wait()

    fire_edges(0, 0)

    def sub(q, p):
        pn = 1 - p

        @pl.when(q < NCHUNKA)
        def _():
            wait_edges(q, p)

        @pl.when(jnp.logical_and(q >= 2, q < NCHUNKA))
        def _():
            wait_scatter(q - 2, p)

        @pl.when(q < NCHUNKA)
        def _():
            mkidx(p)
            fire_gathers(q, p)

        @pl.when(q + 1 < NCHUNKA)
        def _():
            fire_edges(q + 1, pn)

        @pl.when(jnp.logical_and(q >= 1, q - 1 < NCHUNKA))
        def _():
            wait_gathers(q - 1, pn)
            scale(pn)
            fire_scatter(q - 1, pn)

    def pair(i, _):
        sub(2 * i, 0)
        sub(2 * i + 1, 1)
        return 0

    lax.fori_loop(0, (NCHUNKA + 2) // 2, pair, 0)
    wait_scatter(NCHUNKA - 2, (NCHUNKA - 2) % 2)
    wait_scatter(NCHUNKA - 1, (NCHUNKA - 1) % 2)
    plsc.subcore_barrier()
    for mi in range((NCH + NS - 1) // NS):
        m = s + NS * mi

        @pl.when(m < NCH)
        def _():
            pltpu.sync_copy(agg_sh.at[pl.ds(m * CHK, CHK)], zbuf)
            pltpu.sync_copy(zbuf, agg_hbm.at[c, m])


@functools.cache
def _agg_call():
    return pl.kernel(
        _agg_body,
        out_type=jax.ShapeDtypeStruct((NC, NCH, CHK, HH), _f32),
        mesh=_sc_mesh(),
        scratch_types=[
            pltpu.VMEM((3, K), _i32),
            pltpu.VMEM((3, K), _i32),
            pltpu.VMEM((K,), _i32),
            pltpu.VMEM((K,), _i32),
            pltpu.VMEM((K,), _i32),
            pltpu.VMEM((K,), _i32),
            pltpu.VMEM((K,), _i32),
            pltpu.VMEM((K,), _i32),
            pltpu.VMEM((K, HH), _f32),
            pltpu.VMEM((K, HH), _f32),
            pltpu.VMEM((K, L), _f32),
            pltpu.VMEM((K, L), _f32),
            pltpu.VMEM((CHK, HH), _f32),
            pltpu.SemaphoreType.DMA,
            pltpu.SemaphoreType.DMA,
            pltpu.SemaphoreType.DMA,
            pltpu.SemaphoreType.DMA,
            pltpu.SemaphoreType.DMA,
            pltpu.SemaphoreType.DMA,
            pltpu.VMEM_SHARED((N, HH), _f32),
        ],
        compiler_params=pltpu.CompilerParams(use_tc_tiling_on_sc=False),
    )


def _prep_body(cnt_ref, nrm_ref):
    total = cnt_ref[0] + cnt_ref[1]
    nr = 1.0 / jnp.maximum(total, 1.0)
    nrm_ref[...] = jnp.broadcast_to(nr[:, :, None], nr.shape + (L,))


def _prep_call(cnt):
    tn = 1000
    return pl.pallas_call(
        _prep_body,
        grid=(N // tn,),
        in_specs=[pl.BlockSpec((NC, tn, R), lambda n: (0, n, 0))],
        out_specs=pl.BlockSpec((tn, R, L), lambda n: (n, 0, 0)),
        out_shape=jax.ShapeDtypeStruct((N, R, L), _f32),
    )(cnt)


def _z_body(x_ref, wl_ref, wr_ref, z_ref):
    x = x_ref[...]
    z_ref[0] = jnp.dot(x, wl_ref[...], preferred_element_type=_f32)
    z_ref[1] = jnp.dot(x, wr_ref[...], preferred_element_type=_f32)


def _z_call(x, wl, wr):
    tn = 400
    return pl.pallas_call(
        _z_body,
        grid=(N // tn,),
        in_specs=[
            pl.BlockSpec((tn, H), lambda n: (n, 0)),
            pl.BlockSpec((H, R * HH), lambda n: (0, 0)),
            pl.BlockSpec((H, R * HH), lambda n: (0, 0)),
        ],
        out_specs=pl.BlockSpec((NC, tn, R * HH), lambda n: (0, n, 0)),
        out_shape=jax.ShapeDtypeStruct((NC, N, R * HH), _f32),
    )(x, wl, wr)


def _comb_body(agg_ref, x_ref, root_ref, b_ref, out_ref, *, relu):
    a = jnp.concatenate([agg_ref[0], agg_ref[1, :, : H - HH]], axis=1)
    y = a + jnp.dot(x_ref[...], root_ref[...], preferred_element_type=_f32)
    y = y + b_ref[...]
    if relu:
        y = jnp.maximum(y, 0.0)
    out_ref[...] = y


def _comb_call(agg, x, root, b, relu):
    tn = 1000
    return pl.pallas_call(
        functools.partial(_comb_body, relu=relu),
        grid=(N // tn,),
        in_specs=[
            pl.BlockSpec((NC, tn, HH), lambda n: (0, n, 0)),
            pl.BlockSpec((tn, H), lambda n: (n, 0)),
            pl.BlockSpec((H, H), lambda n: (0, 0)),
            pl.BlockSpec((1, H), lambda n: (0, 0)),
        ],
        out_specs=pl.BlockSpec((tn, H), lambda n: (n, 0)),
        out_shape=jax.ShapeDtypeStruct((N, H), _f32),
    )(agg, x, root, b)


def _expand_weights(w):
    wfull = jnp.zeros((R, H, HP), _f32)
    for bidx in range(NB):
        lo = bidx * HB
        wfull = wfull.at[:, lo:lo + HB, lo:lo + HB].set(w[:, bidx])
    wl = wfull[:, :, :HH].transpose(1, 0, 2).reshape(H, R * HH)
    wr = wfull[:, :, HH:].transpose(1, 0, 2).reshape(H, R * HH)
    return wl, wr


def _layer(x, ei, wl, wr, root, b, normrep, relu):
    z = _z_call(x, wl, wr).reshape(NC * N * R, HH)
    agg = _agg_call()(ei, z, normrep).reshape(NC, N, HH)
    return _comb_call(agg, x, root, b.reshape(1, H), relu)


def kernel(edge_index, edge_type, node_emb, w1, root1, b1, w2, root2, b2):
    ei = jnp.stack([edge_index[0], edge_index[1], edge_type])
    eye = jnp.eye(R, L, dtype=_f32)
    cnt = _counts_call()(ei, eye).reshape(NC, N, R)
    normrep = _prep_call(cnt).reshape(N * R, L)
    wl1, wr1 = _expand_weights(w1)
    wl2, wr2 = _expand_weights(w2)
    x1 = _layer(node_emb, ei, wl1, wr1, root1, b1, normrep, True)
    out = _layer(x1, ei, wl2, wr2, root2, b2, normrep, False)
    return out


# speedup vs baseline: 41.8200x; 1.5183x over previous
import functools

import jax
import jax.numpy as jnp
from jax import lax
from jax.experimental import pallas as pl
from jax.experimental.pallas import tpu as pltpu
from jax.experimental.pallas import tpu_sc as plsc

N = 10000
H = 200
R = 16
NB = 5
HB = H // NB
E = 320000
L = 16
HP = 224
HH = 112
NVH = HH // L
NC = 2
NS = 16
EC = E // NC
ET = EC // NS
ETA = E // NS
K = 80
NCHUNK = ET // K
NCHUNKA = ETA // K
CHK = 200
NCH = N // CHK

_f32 = jnp.float32
_i32 = jnp.int32

@functools.cache
def _sc_mesh():
    return plsc.VectorSubcoreMesh(
        core_axis_name="c", subcore_axis_name="s", num_cores=NC, num_subcores=NS
    )


def _counts_body(ei_hbm, eye_hbm, cnt_hbm, eb0, eb1, tp0, tp1, dt0, dt1,
                 oh0, oh1, obuf, semE0, semE1, semG0, semG1, semS0, semS1,
                 cnt_sh, eye_sh):
    c = lax.axis_index("c")
    s = lax.axis_index("s")
    ebufs = (eb0, eb1)
    tps = (tp0, tp1)
    dts = (dt0, dt1)
    ohs = (oh0, oh1)
    semE = (semE0, semE1)
    semG = (semG0, semG1)
    semS = (semS0, semS1)

    @pl.when(s == 0)
    def _():
        pltpu.sync_copy(eye_hbm, obuf.at[pl.ds(0, R)])
        pltpu.sync_copy(obuf.at[pl.ds(0, R)], eye_sh)

    def zrow(i, _):
        obuf[i, :] = jnp.zeros((L,), _f32)
        return 0

    lax.fori_loop(0, CHK, zrow, 0)
    for mi in range((NCH + NS - 1) // NS):
        m = s + NS * mi

        @pl.when(m < NCH)
        def _():
            pltpu.sync_copy(obuf, cnt_sh.at[pl.ds(m * CHK, CHK)])

    plsc.subcore_barrier()

    base0 = c * EC + s * ET

    def esrc(q):
        return ei_hbm.at[:, pl.ds(base0 + q * K, K)]

    def fire_edges(q, p):
        pltpu.async_copy(esrc(q), ebufs[p], semE[p])

    def wait_edges(q, p):
        pltpu.make_async_copy(esrc(q), ebufs[p], semE[p]).wait()

    def copy_idx(p):
        eb, tp, dt = ebufs[p], tps[p], dts[p]

        def one(i, _):
            dt[pl.ds(L * i, L)] = eb[1, pl.ds(L * i, L)]
            tp[pl.ds(L * i, L)] = eb[2, pl.ds(L * i, L)]
            return 0

        lax.fori_loop(0, K // L, one, 0)

    def fire_oh(q, p):
        pltpu.async_copy(eye_sh.at[tps[p]], ohs[p], semG[p])

    def wait_oh(q, p):
        pltpu.make_async_copy(eye_sh.at[tps[p]], ohs[p], semG[p]).wait()

    def fire_scatter(q, p):
        pltpu.async_copy(ohs[p], cnt_sh.at[dts[p]], semS[p], add=True)

    def wait_scatter(q, p):
        pltpu.make_async_copy(ohs[p], cnt_sh.at[dts[p]], semS[p]).wait()

    fire_edges(0, 0)

    def sub(q, p):
        pn = 1 - p

        @pl.when(q < NCHUNK)
        def _():
            wait_edges(q, p)

        @pl.when(jnp.logical_and(q >= 2, q < NCHUNK))
        def _():
            wait_scatter(q - 2, p)

        @pl.when(q < NCHUNK)
        def _():
            copy_idx(p)
            fire_oh(q, p)

        @pl.when(q + 1 < NCHUNK)
        def _():
            fire_edges(q + 1, pn)

        @pl.when(jnp.logical_and(q >= 1, q - 1 < NCHUNK))
        def _():
            wait_oh(q - 1, pn)
            fire_scatter(q - 1, pn)

    def pair(i, _):
        sub(2 * i, 0)
        sub(2 * i + 1, 1)
        return 0

    lax.fori_loop(0, (NCHUNK + 2) // 2, pair, 0)
    wait_scatter(NCHUNK - 2, (NCHUNK - 2) % 2)
    wait_scatter(NCHUNK - 1, (NCHUNK - 1) % 2)
    plsc.subcore_barrier()
    for mi in range((NCH + NS - 1) // NS):
        m = s + NS * mi

        @pl.when(m < NCH)
        def _():
            pltpu.sync_copy(cnt_sh.at[pl.ds(m * CHK, CHK)], obuf)
            pltpu.sync_copy(obuf, cnt_hbm.at[c, m])


@functools.cache
def _counts_call():
    return pl.kernel(
        _counts_body,
        out_type=jax.ShapeDtypeStruct((NC, NCH, CHK, R), _f32),
        mesh=_sc_mesh(),
        scratch_types=[
            pltpu.VMEM((3, K), _i32),
            pltpu.VMEM((3, K), _i32),
            pltpu.VMEM((K,), _i32),
            pltpu.VMEM((K,), _i32),
            pltpu.VMEM((K,), _i32),
            pltpu.VMEM((K,), _i32),
            pltpu.VMEM((K, R), _f32),
            pltpu.VMEM((K, R), _f32),
            pltpu.VMEM((CHK, R), _f32),
            pltpu.SemaphoreType.DMA,
            pltpu.SemaphoreType.DMA,
            pltpu.SemaphoreType.DMA,
            pltpu.SemaphoreType.DMA,
            pltpu.SemaphoreType.DMA,
            pltpu.SemaphoreType.DMA,
            pltpu.VMEM_SHARED((N, R), _f32),
            pltpu.VMEM_SHARED((R, L), _f32),
        ],
        compiler_params=pltpu.CompilerParams(use_tc_tiling_on_sc=False),
    )


def _agg_body(ei_hbm, z_hbm, nrm_hbm, agg_hbm,
              eb0, eb1, zx0, zx1, sg0, sg1, dv0, dv1, rw0, rw1, nm0, nm1,
              zbuf, semE0, semE1, semG0, semG1, semS0, semS1, agg_sh):
    c = lax.axis_index("c")
    s = lax.axis_index("s")
    ebufs = (eb0, eb1)
    zidxs = (zx0, zx1)
    segvs = (sg0, sg1)
    dsts = (dv0, dv1)
    rws = (rw0, rw1)
    nms = (nm0, nm1)
    semE = (semE0, semE1)
    semG = (semG0, semG1)
    semS = (semS0, semS1)

    def zrow(i, _):
        for j in range(NVH):
            zbuf[i, pl.ds(L * j, L)] = jnp.zeros((L,), _f32)
        return 0

    lax.fori_loop(0, CHK, zrow, 0)
    for mi in range((NCH + NS - 1) // NS):
        m = s + NS * mi

        @pl.when(m < NCH)
        def _():
            pltpu.sync_copy(zbuf, agg_sh.at[pl.ds(m * CHK, CHK)])

    plsc.subcore_barrier()

    base0 = s * ETA
    zoff = c * (N * R)

    def esrc(q):
        return ei_hbm.at[:, pl.ds(base0 + q * K, K)]

    def fire_edges(q, p):
        pltpu.async_copy(esrc(q), ebufs[p], semE[p])

    def wait_edges(q, p):
        pltpu.make_async_copy(esrc(q), ebufs[p], semE[p]).wait()

    def mkidx(p):
        eb, zidx, segv, dstv = ebufs[p], zidxs[p], segvs[p], dsts[p]

        def one(i, _):
            sv = eb[0, pl.ds(L * i, L)]
            dv = eb[1, pl.ds(L * i, L)]
            tv = eb[2, pl.ds(L * i, L)]
            zidx[pl.ds(L * i, L)] = zoff + sv * R + tv
            segv[pl.ds(L * i, L)] = dv * R + tv
            dstv[pl.ds(L * i, L)] = dv
            return 0

        lax.fori_loop(0, K // L, one, 0)

    def fire_gathers(q, p):
        pltpu.async_copy(z_hbm.at[zidxs[p]], rws[p], semG[p])
        pltpu.async_copy(nrm_hbm.at[segvs[p]], nms[p], semG[p])

    def wait_gathers(q, p):
        pltpu.make_async_copy(z_hbm.at[zidxs[p]], rws[p], semG[p]).wait()
        pltpu.make_async_copy(nrm_hbm.at[segvs[p]], nms[p], semG[p]).wait()

    def scale(p):
        rows, nrmv = rws[p], nms[p]

        def one(k, _):
            b = nrmv[k, :]
            for j in range(NVH):
                rows[k, pl.ds(L * j, L)] = rows[k, pl.ds(L * j, L)] * b
            return 0

        lax.fori_loop(0, K, one, 0)

    def fire_scatter(q, p):
        pltpu.async_copy(rws[p], agg_sh.at[dsts[p]], semS[p], add=True)

    def wait_scatter(q, p):
        pltpu.make_async_copy(rws[p], agg_sh.at[dsts[p]], semS[p]).wait()

    fire_edges(0, 0)

    def sub(q, p):
        pn = 1 - p

        @pl.when(q < NCHUNKA)
        def _():
            wait_edges(q, p)

        @pl.when(jnp.logical_and(q >= 2, q < NCHUNKA))
        def _():
            wait_scatter(q - 2, p)

        @pl.when(q < NCHUNKA)
        def _():
            mkidx(p)
            fire_gathers(q, p)

        @pl.when(q + 1 < NCHUNKA)
        def _():
            fire_edges(q + 1, pn)

        @pl.when(jnp.logical_and(q >= 1, q - 1 < NCHUNKA))
        def _():
            wait_gathers(q - 1, pn)
            scale(pn)
            fire_scatter(q - 1, pn)

    def pair(i, _):
        sub(2 * i, 0)
        sub(2 * i + 1, 1)
        return 0

    lax.fori_loop(0, (NCHUNKA + 2) // 2, pair, 0)
    wait_scatter(NCHUNKA - 2, (NCHUNKA - 2) % 2)
    wait_scatter(NCHUNKA - 1, (NCHUNKA - 1) % 2)
    plsc.subcore_barrier()
    for mi in range((NCH + NS - 1) // NS):
        m = s + NS * mi

        @pl.when(m < NCH)
        def _():
            pltpu.sync_copy(agg_sh.at[pl.ds(m * CHK, CHK)], zbuf)
            pltpu.sync_copy(zbuf, agg_hbm.at[c, m])


@functools.cache
def _agg_call():
    return pl.kernel(
        _agg_body,
        out_type=jax.ShapeDtypeStruct((NC, NCH, CHK, HH), _f32),
        mesh=_sc_mesh(),
        scratch_types=[
            pltpu.VMEM((3, K), _i32),
            pltpu.VMEM((3, K), _i32),
            pltpu.VMEM((K,), _i32),
            pltpu.VMEM((K,), _i32),
            pltpu.VMEM((K,), _i32),
            pltpu.VMEM((K,), _i32),
            pltpu.VMEM((K,), _i32),
            pltpu.VMEM((K,), _i32),
            pltpu.VMEM((K, HH), _f32),
            pltpu.VMEM((K, HH), _f32),
            pltpu.VMEM((K, L), _f32),
            pltpu.VMEM((K, L), _f32),
            pltpu.VMEM((CHK, HH), _f32),
            pltpu.SemaphoreType.DMA,
            pltpu.SemaphoreType.DMA,
            pltpu.SemaphoreType.DMA,
            pltpu.SemaphoreType.DMA,
            pltpu.SemaphoreType.DMA,
            pltpu.SemaphoreType.DMA,
            pltpu.VMEM_SHARED((N, HH), _f32),
        ],
        compiler_params=pltpu.CompilerParams(use_tc_tiling_on_sc=False),
    )


def _prep_body(cnt_ref, nrm_ref):
    total = cnt_ref[0] + cnt_ref[1]
    nr = 1.0 / jnp.maximum(total, 1.0)
    nrm_ref[...] = jnp.broadcast_to(nr[:, :, None], nr.shape + (L,))


def _prep_call(cnt):
    tn = 1000
    return pl.pallas_call(
        _prep_body,
        grid=(N // tn,),
        in_specs=[pl.BlockSpec((NC, tn, R), lambda n: (0, n, 0))],
        out_specs=pl.BlockSpec((tn, R, L), lambda n: (n, 0, 0)),
        out_shape=jax.ShapeDtypeStruct((N, R, L), _f32),
    )(cnt)


def _z_body(x_ref, wl_ref, wr_ref, z_ref):
    x = x_ref[...]
    z_ref[0] = jnp.dot(x, wl_ref[...], preferred_element_type=_f32)
    z_ref[1] = jnp.dot(x, wr_ref[...], preferred_element_type=_f32)


def _z_call(x, wl, wr):
    tn = 400
    return pl.pallas_call(
        _z_body,
        grid=(N // tn,),
        in_specs=[
            pl.BlockSpec((tn, H), lambda n: (n, 0)),
            pl.BlockSpec((H, R * HH), lambda n: (0, 0)),
            pl.BlockSpec((H, R * HH), lambda n: (0, 0)),
        ],
        out_specs=pl.BlockSpec((NC, tn, R * HH), lambda n: (0, n, 0)),
        out_shape=jax.ShapeDtypeStruct((NC, N, R * HH), _f32),
    )(x, wl, wr)


def _comb_body(agg_ref, x_ref, root_ref, b_ref, out_ref, *, relu):
    a = jnp.concatenate([agg_ref[0], agg_ref[1, :, : H - HH]], axis=1)
    y = a + jnp.dot(x_ref[...], root_ref[...], preferred_element_type=_f32)
    y = y + b_ref[...]
    if relu:
        y = jnp.maximum(y, 0.0)
    out_ref[...] = y


def _comb_call(agg, x, root, b, relu):
    tn = 1000
    return pl.pallas_call(
        functools.partial(_comb_body, relu=relu),
        grid=(N // tn,),
        in_specs=[
            pl.BlockSpec((NC, tn, HH), lambda n: (0, n, 0)),
            pl.BlockSpec((tn, H), lambda n: (n, 0)),
            pl.BlockSpec((H, H), lambda n: (0, 0)),
            pl.BlockSpec((1, H), lambda n: (0, 0)),
        ],
        out_specs=pl.BlockSpec((tn, H), lambda n: (n, 0)),
        out_shape=jax.ShapeDtypeStruct((N, H), _f32),
    )(agg, x, root, b)


def _expand_weights(w):
    wfull = jnp.zeros((R, H, HP), _f32)
    for bidx in range(NB):
        lo = bidx * HB
        wfull = wfull.at[:, lo:lo + HB, lo:lo + HB].set(w[:, bidx])
    wl = wfull[:, :, :HH].transpose(1, 0, 2).reshape(H, R * HH)
    wr = wfull[:, :, HH:].transpose(1, 0, 2).reshape(H, R * HH)
    return wl, wr


def _layer(x, ei, wl, wr, root, b, normrep, relu):
    z = _z_call(x, wl, wr).reshape(NC * N * R, HH)
    agg = _agg_call()(ei, z, normrep).reshape(NC, N, HH)
    return _comb_call(agg, x, root, b.reshape(1, H), relu)


def kernel(edge_index, edge_type, node_emb, w1, root1, b1, w2, root2, b2):
    ei = jnp.stack([edge_index[0], edge_index[1], edge_type])
    eye = jnp.eye(R, L, dtype=_f32)
    cnt = _counts_call()(ei, eye).reshape(NC, N, R)
    normrep = _prep_call(cnt).reshape(N * R, L)
    wl1, wr1 = _expand_weights(w1)
    wl2, wr2 = _expand_weights(w2)
    x1 = _layer(node_emb, ei, wl1, wr1, root1, b1, normrep, True)
    out = _layer(x1, ei, wl2, wr2, root2, b2, normrep, False)
    return out


# speedup vs baseline: 47.7660x; 1.1422x over previous
import functools

import jax
import jax.numpy as jnp
from jax import lax
from jax.experimental import pallas as pl
from jax.experimental.pallas import tpu as pltpu
from jax.experimental.pallas import tpu_sc as plsc

N = 10000
H = 200
R = 16
NB = 5
HB = H // NB
E = 320000
L = 16
HH = 128
NVH = HH // L
NC = 2
NS = 16
EC = E // NC
ET = EC // NS
ETA = E // NS
K = 80
NCHUNK = ET // K
NCHUNKA = ETA // K
CHK = 80
NCH = N // CHK

_f32 = jnp.float32
_i32 = jnp.int32


@functools.cache
def _sc_mesh():
    return plsc.VectorSubcoreMesh(
        core_axis_name="c", subcore_axis_name="s", num_cores=NC, num_subcores=NS
    )


def _counts_body(ei_hbm, eye_hbm, cnt_hbm, eb0, eb1, tp0, tp1, dt0, dt1,
                 oh0, oh1, obuf, semE0, semE1, semG0, semG1, semS0, semS1,
                 cnt_sh, eye_sh):
    c = lax.axis_index("c")
    s = lax.axis_index("s")
    ebufs = (eb0, eb1)
    tps = (tp0, tp1)
    dts = (dt0, dt1)
    ohs = (oh0, oh1)
    semE = (semE0, semE1)
    semG = (semG0, semG1)
    semS = (semS0, semS1)

    @pl.when(s == 0)
    def _():
        pltpu.sync_copy(eye_hbm, obuf.at[pl.ds(0, R)])
        pltpu.sync_copy(obuf.at[pl.ds(0, R)], eye_sh)

    def zrow(i, _):
        obuf[i, :] = jnp.zeros((L,), _f32)
        return 0

    lax.fori_loop(0, CHK, zrow, 0)
    for mi in range((NCH + NS - 1) // NS):
        m = s + NS * mi

        @pl.when(m < NCH)
        def _():
            pltpu.sync_copy(obuf, cnt_sh.at[pl.ds(m * CHK, CHK)])

    plsc.subcore_barrier()

    base0 = c * EC + s * ET

    def esrc(q):
        return ei_hbm.at[:, pl.ds(base0 + q * K, K)]

    def fire_edges(q, p):
        pltpu.async_copy(esrc(q), ebufs[p], semE[p])

    def wait_edges(q, p):
        pltpu.make_async_copy(esrc(q), ebufs[p], semE[p]).wait()

    def copy_idx(p):
        eb, tp, dt = ebufs[p], tps[p], dts[p]

        def one(i, _):
            dt[pl.ds(L * i, L)] = eb[1, pl.ds(L * i, L)]
            tp[pl.ds(L * i, L)] = eb[2, pl.ds(L * i, L)]
            return 0

        lax.fori_loop(0, K // L, one, 0)

    def fire_oh(q, p):
        pltpu.async_copy(eye_sh.at[tps[p]], ohs[p], semG[p])

    def wait_oh(q, p):
        pltpu.make_async_copy(eye_sh.at[tps[p]], ohs[p], semG[p]).wait()

    def fire_scatter(q, p):
        pltpu.async_copy(ohs[p], cnt_sh.at[dts[p]], semS[p], add=True)

    def wait_scatter(q, p):
        pltpu.make_async_copy(ohs[p], cnt_sh.at[dts[p]], semS[p]).wait()

    fire_edges(0, 0)

    def sub(q, p):
        pn = 1 - p

        @pl.when(q < NCHUNK)
        def _():
            wait_edges(q, p)

        @pl.when(jnp.logical_and(q >= 2, q < NCHUNK))
        def _():
            wait_scatter(q - 2, p)

        @pl.when(q < NCHUNK)
        def _():
            copy_idx(p)
            fire_oh(q, p)

        @pl.when(q + 1 < NCHUNK)
        def _():
            fire_edges(q + 1, pn)

        @pl.when(jnp.logical_and(q >= 1, q - 1 < NCHUNK))
        def _():
            wait_oh(q - 1, pn)
            fire_scatter(q - 1, pn)

    def pair(i, _):
        sub(2 * i, 0)
        sub(2 * i + 1, 1)
        return 0

    lax.fori_loop(0, (NCHUNK + 2) // 2, pair, 0)
    wait_scatter(NCHUNK - 2, (NCHUNK - 2) % 2)
    wait_scatter(NCHUNK - 1, (NCHUNK - 1) % 2)
    plsc.subcore_barrier()
    for mi in range((NCH + NS - 1) // NS):
        m = s + NS * mi

        @pl.when(m < NCH)
        def _():
            pltpu.sync_copy(cnt_sh.at[pl.ds(m * CHK, CHK)], obuf)
            pltpu.sync_copy(obuf, cnt_hbm.at[c, m])


@functools.cache
def _counts_call():
    return pl.kernel(
        _counts_body,
        out_type=jax.ShapeDtypeStruct((NC, NCH, CHK, R), _f32),
        mesh=_sc_mesh(),
        scratch_types=[
            pltpu.VMEM((3, K), _i32),
            pltpu.VMEM((3, K), _i32),
            pltpu.VMEM((K,), _i32),
            pltpu.VMEM((K,), _i32),
            pltpu.VMEM((K,), _i32),
            pltpu.VMEM((K,), _i32),
            pltpu.VMEM((K, R), _f32),
            pltpu.VMEM((K, R), _f32),
            pltpu.VMEM((CHK, R), _f32),
            pltpu.SemaphoreType.DMA,
            pltpu.SemaphoreType.DMA,
            pltpu.SemaphoreType.DMA,
            pltpu.SemaphoreType.DMA,
            pltpu.SemaphoreType.DMA,
            pltpu.SemaphoreType.DMA,
            pltpu.VMEM_SHARED((N, R), _f32),
            pltpu.VMEM_SHARED((R, L), _f32),
        ],
        compiler_params=pltpu.CompilerParams(use_tc_tiling_on_sc=False),
    )


def _norme_body(ei_hbm, nrm_hbm, ne_hbm, eb0, eb1, sg0, sg1, nb0, nb1,
                semE0, semE1, semG0, semG1, semS0, semS1):
    c = lax.axis_index("c")
    s = lax.axis_index("s")
    ebufs = (eb0, eb1)
    segvs = (sg0, sg1)
    nbs = (nb0, nb1)
    semE = (semE0, semE1)
    semG = (semG0, semG1)
    semS = (semS0, semS1)

    base0 = c * EC + s * ET

    def esrc(q):
        return ei_hbm.at[:, pl.ds(base0 + q * K, K)]

    def ndst(q):
        return ne_hbm.at[pl.ds(base0 + q * K, K)]

    def fire_edges(q, p):
        pltpu.async_copy(esrc(q), ebufs[p], semE[p])

    def wait_edges(q, p):
        pltpu.make_async_copy(esrc(q), ebufs[p], semE[p]).wait()

    def mkseg(p):
        eb, segv = ebufs[p], segvs[p]

        def one(i, _):
            dv = eb[1, pl.ds(L * i, L)]
            tv = eb[2, pl.ds(L * i, L)]
            segv[pl.ds(L * i, L)] = dv * R + tv
            return 0

        lax.fori_loop(0, K // L, one, 0)

    def fire_gather(q, p):
        pltpu.async_copy(nrm_hbm.at[segvs[p]], nbs[p], semG[p])

    def wait_gather(q, p):
        pltpu.make_async_copy(nrm_hbm.at[segvs[p]], nbs[p], semG[p]).wait()

    def fire_store(q, p):
        pltpu.async_copy(nbs[p], ndst(q), semS[p])

    def wait_store(q, p):
        pltpu.make_async_copy(nbs[p], ndst(q), semS[p]).wait()

    fire_edges(0, 0)

    def sub(q, p):
        pn = 1 - p

        @pl.when(q < NCHUNK)
        def _():
            wait_edges(q, p)

        @pl.when(jnp.logical_and(q >= 2, q < NCHUNK))
        def _():
            wait_store(q - 2, p)

        @pl.when(q < NCHUNK)
        def _():
            mkseg(p)
            fire_gather(q, p)

        @pl.when(q + 1 < NCHUNK)
        def _():
            fire_edges(q + 1, pn)

        @pl.when(jnp.logical_and(q >= 1, q - 1 < NCHUNK))
        def _():
            wait_gather(q - 1, pn)
            fire_store(q - 1, pn)

    def pair(i, _):
        sub(2 * i, 0)
        sub(2 * i + 1, 1)
        return 0

    lax.fori_loop(0, (NCHUNK + 2) // 2, pair, 0)
    wait_store(NCHUNK - 2, (NCHUNK - 2) % 2)
    wait_store(NCHUNK - 1, (NCHUNK - 1) % 2)


@functools.cache
def _norme_call():
    return pl.kernel(
        _norme_body,
        out_type=jax.ShapeDtypeStruct((E, L), _f32),
        mesh=_sc_mesh(),
        scratch_types=[
            pltpu.VMEM((3, K), _i32),
            pltpu.VMEM((3, K), _i32),
            pltpu.VMEM((K,), _i32),
            pltpu.VMEM((K,), _i32),
            pltpu.VMEM((K, L), _f32),
            pltpu.VMEM((K, L), _f32),
            pltpu.SemaphoreType.DMA,
            pltpu.SemaphoreType.DMA,
            pltpu.SemaphoreType.DMA,
            pltpu.SemaphoreType.DMA,
            pltpu.SemaphoreType.DMA,
            pltpu.SemaphoreType.DMA,
        ],
        compiler_params=pltpu.CompilerParams(use_tc_tiling_on_sc=False),
    )


def _agg_body(src_hbm, dst_hbm, typ_hbm, z_hbm, ne_hbm, agg_hbm,
              eb0, eb1, zx0, zx1, dv0, dv1, rw0, rw1, nm0, nm1,
              zbuf, semE0, semE1, semG0, semG1, semS0, semS1, agg_sh):
    c = lax.axis_index("c")
    s = lax.axis_index("s")
    ebufs = (eb0, eb1)
    zidxs = (zx0, zx1)
    dsts = (dv0, dv1)
    rws = (rw0, rw1)
    nms = (nm0, nm1)
    semE = (semE0, semE1)
    semG = (semG0, semG1)
    semS = (semS0, semS1)

    def zrow(i, _):
        for j in range(NVH):
            zbuf[i, pl.ds(L * j, L)] = jnp.zeros((L,), _f32)
        return 0

    lax.fori_loop(0, CHK, zrow, 0)
    for mi in range((NCH + NS - 1) // NS):
        m = s + NS * mi

        @pl.when(m < NCH)
        def _():
            pltpu.sync_copy(zbuf, agg_sh.at[pl.ds(m * CHK, CHK)])

    plsc.subcore_barrier()

    base0 = s * ETA
    zoff = c * (R * N)

    def base(q):
        return base0 + q * K

    def fire_edges(q, p):
        b = base(q)
        pltpu.async_copy(src_hbm.at[pl.ds(b, K)], ebufs[p].at[0], semE[p])
        pltpu.async_copy(dst_hbm.at[pl.ds(b, K)], ebufs[p].at[1], semE[p])
        pltpu.async_copy(typ_hbm.at[pl.ds(b, K)], ebufs[p].at[2], semE[p])
        pltpu.async_copy(ne_hbm.at[pl.ds(b * L, K * L)], nms[p], semE[p])

    def wait_edges(q, p):
        b = base(q)
        pltpu.make_async_copy(src_hbm.at[pl.ds(b, K)], ebufs[p].at[0],
                              semE[p]).wait()
        pltpu.make_async_copy(dst_hbm.at[pl.ds(b, K)], ebufs[p].at[1],
                              semE[p]).wait()
        pltpu.make_async_copy(typ_hbm.at[pl.ds(b, K)], ebufs[p].at[2],
                              semE[p]).wait()
        pltpu.make_async_copy(ne_hbm.at[pl.ds(b * L, K * L)], nms[p],
                              semE[p]).wait()

    def mkidx(p):
        eb, zidx, dstv = ebufs[p], zidxs[p], dsts[p]

        def one(i, _):
            sv = eb[0, pl.ds(L * i, L)]
            dv = eb[1, pl.ds(L * i, L)]
            tv = eb[2, pl.ds(L * i, L)]
            zidx[pl.ds(L * i, L)] = zoff + tv * N + sv
            dstv[pl.ds(L * i, L)] = dv
            return 0

        lax.fori_loop(0, K // L, one, 0)

    def fire_gather(q, p):
        pltpu.async_copy(z_hbm.at[zidxs[p]], rws[p], semG[p])

    def wait_gather(q, p):
        pltpu.make_async_copy(z_hbm.at[zidxs[p]], rws[p], semG[p]).wait()

    def scale(p):
        rows, nrmv = rws[p], nms[p]

        def one(k, _):
            b = nrmv[pl.ds(L * k, L)]
            for j in range(NVH):
                rows[k, pl.ds(L * j, L)] = rows[k, pl.ds(L * j, L)] * b
            return 0

        lax.fori_loop(0, K, one, 0)

    def fire_scatter(q, p):
        pltpu.async_copy(rws[p], agg_sh.at[dsts[p]], semS[p], add=True)

    def wait_scatter(q, p):
        pltpu.make_async_copy(rws[p], agg_sh.at[dsts[p]], semS[p]).wait()

    fire_edges(0, 0)

    def sub(q, p):
        pn = 1 - p

        @pl.when(q < NCHUNKA)
        def _():
            wait_edges(q, p)

        @pl.when(jnp.logical_and(q >= 2, q < NCHUNKA))
        def _():
            wait_scatter(q - 2, p)

        @pl.when(q < NCHUNKA)
        def _():
            mkidx(p)
            fire_gather(q, p)

        @pl.when(q + 1 < NCHUNKA)
        def _():
            fire_edges(q + 1, pn)

        @pl.when(jnp.logical_and(q >= 1, q - 1 < NCHUNKA))
        def _():
            wait_gather(q - 1, pn)
            scale(pn)
            fire_scatter(q - 1, pn)

    def pair(i, _):
        sub(2 * i, 0)
        sub(2 * i + 1, 1)
        return 0

    lax.fori_loop(0, (NCHUNKA + 2) // 2, pair, 0)
    wait_scatter(NCHUNKA - 2, (NCHUNKA - 2) % 2)
    wait_scatter(NCHUNKA - 1, (NCHUNKA - 1) % 2)
    plsc.subcore_barrier()
    for mi in range((NCH + NS - 1) // NS):
        m = s + NS * mi

        @pl.when(m < NCH)
        def _():
            pltpu.sync_copy(agg_sh.at[pl.ds(m * CHK, CHK)], zbuf)
            pltpu.sync_copy(zbuf, agg_hbm.at[c, m])


@functools.cache
def _agg_call():
    return pl.kernel(
        _agg_body,
        out_type=jax.ShapeDtypeStruct((NC, NCH, CHK, HH), _f32),
        mesh=_sc_mesh(),
        scratch_types=[
            pltpu.VMEM((3, K), _i32),
            pltpu.VMEM((3, K), _i32),
            pltpu.VMEM((K,), _i32),
            pltpu.VMEM((K,), _i32),
            pltpu.VMEM((K,), _i32),
            pltpu.VMEM((K,), _i32),
            pltpu.VMEM((K, HH), _f32),
            pltpu.VMEM((K, HH), _f32),
            pltpu.VMEM((K * L,), _f32),
            pltpu.VMEM((K * L,), _f32),
            pltpu.VMEM((CHK, HH), _f32),
            pltpu.SemaphoreType.DMA,
            pltpu.SemaphoreType.DMA,
            pltpu.SemaphoreType.DMA,
            pltpu.SemaphoreType.DMA,
            pltpu.SemaphoreType.DMA,
            pltpu.SemaphoreType.DMA,
            pltpu.VMEM_SHARED((N, HH), _f32),
        ],
    )


def _prep_body(cnt_ref, nrm_ref):
    total = cnt_ref[0] + cnt_ref[1]
    nr = 1.0 / jnp.maximum(total, 1.0)
    nrm_ref[...] = jnp.broadcast_to(nr[:, :, None], nr.shape + (L,))


def _prep_call(cnt):
    tn = 1000
    return pl.pallas_call(
        _prep_body,
        grid=(N // tn,),
        in_specs=[pl.BlockSpec((NC, tn, R), lambda n: (0, n, 0))],
        out_specs=pl.BlockSpec((tn, R, L), lambda n: (n, 0, 0)),
        out_shape=jax.ShapeDtypeStruct((N, R, L), _f32),
    )(cnt)


def _z_body(x_ref, w_ref, z_ref):
    x = x_ref[...]
    for h in range(NC):
        for r in range(R):
            z_ref[h, r] = jnp.dot(x, w_ref[h, r],
                                  preferred_element_type=_f32)


def _z_call(x, wsplit):
    tn = 400
    return pl.pallas_call(
        _z_body,
        grid=(N // tn,),
        in_specs=[
            pl.BlockSpec((tn, H), lambda n: (n, 0)),
            pl.BlockSpec((NC, R, H, HH), lambda n: (0, 0, 0, 0)),
        ],
        out_specs=pl.BlockSpec((NC, R, tn, HH), lambda n: (0, 0, n, 0)),
        out_shape=jax.ShapeDtypeStruct((NC, R, N, HH), _f32),
    )(x, wsplit)


def _comb_body(agg_ref, x_ref, root_ref, b_ref, out_ref, *, relu):
    a = jnp.concatenate([agg_ref[0], agg_ref[1, :, : H - HH]], axis=1)
    y = a + jnp.dot(x_ref[...], root_ref[...], preferred_element_type=_f32)
    y = y + b_ref[...]
    if relu:
        y = jnp.maximum(y, 0.0)
    out_ref[...] = y


def _comb_call(agg, x, root, b, relu):
    tn = 1000
    return pl.pallas_call(
        functools.partial(_comb_body, relu=relu),
        grid=(N // tn,),
        in_specs=[
            pl.BlockSpec((NC, tn, HH), lambda n: (0, n, 0)),
            pl.BlockSpec((tn, H), lambda n: (n, 0)),
            pl.BlockSpec((H, H), lambda n: (0, 0)),
            pl.BlockSpec((1, H), lambda n: (0, 0)),
        ],
        out_specs=pl.BlockSpec((tn, H), lambda n: (n, 0)),
        out_shape=jax.ShapeDtypeStruct((N, H), _f32),
    )(agg, x, root, b)


def _expand_weights(w):
    wfull = jnp.zeros((R, H, NC * HH), _f32)
    for bidx in range(NB):
        lo = bidx * HB
        wfull = wfull.at[:, lo:lo + HB, lo:lo + HB].set(w[:, bidx])
    return jnp.stack([wfull[:, :, :HH], wfull[:, :, HH:]])


def _layer(x, src, dst, typ, wsplit, root, b, ne, relu):
    z = _z_call(x, wsplit).reshape(NC * R * N, HH)
    agg = _agg_call()(src, dst, typ, z, ne).reshape(NC, N, HH)
    return _comb_call(agg, x, root, b.reshape(1, H), relu)


def kernel(edge_index, edge_type, node_emb, w1, root1, b1, w2, root2, b2):
    src = edge_index[0]
    dst = edge_index[1]
    typ = edge_type
    ei = jnp.stack([src, dst, typ])
    eye = jnp.eye(R, L, dtype=_f32)
    cnt = _counts_call()(ei, eye).reshape(NC, N, R)
    normrep = _prep_call(cnt).reshape(N * R, L)
    ne = _norme_call()(ei, normrep).reshape(E * L)
    ws1 = _expand_weights(w1)
    ws2 = _expand_weights(w2)
    x1 = _layer(node_emb, src, dst, typ, ws1, root1, b1, ne, True)
    out = _layer(x1, src, dst, typ, ws2, root2, b2, ne, False)
    return out
